# Initial kernel scaffold; baseline (speedup 1.0000x reference)
import jax
import jax.numpy as jnp
from jax.experimental import pallas as pl


def kernel(q_key, q_selection, mem_key, mem_shrinkage, mem_value):
    raise NotImplementedError("write your pallas kernel here")


# speedup vs baseline: 7.6261x; 7.6261x over previous
import functools
import math

import jax
import jax.numpy as jnp
from jax.experimental import pallas as pl
from jax.experimental.pallas import tpu as pltpu

H = 32
W = 32
HW = H * W
T = 16384
CK = 64
CV = 512
TOP_K = 30

QB = 128
NEG = -3.0e38


def _affinity_body(qk_ref, qs_ref, mkT_ref, mk2T_ref, shr_ref, e_ref):
    qk = qk_ref[...]
    qs = qs_ref[...]
    mkT = mkT_ref[...]
    mk2T = mk2T_ref[...]
    shr = shr_ref[...]

    a_sq = jnp.dot(mk2T, qs, preferred_element_type=jnp.float32)
    two_ab = 2.0 * jnp.dot(mkT, qk * qs, preferred_element_type=jnp.float32)
    b_sq = jnp.sum(qs * (qk * qk), axis=0, keepdims=True)
    sim = (-a_sq + two_ab - b_sq) * shr * (1.0 / math.sqrt(CK))

    thr = jnp.max(sim, axis=0, keepdims=True)

    def step(_, th):
        return jnp.max(jnp.where(sim < th, sim, NEG), axis=0, keepdims=True)

    thr = jax.lax.fori_loop(0, TOP_K - 1, step, thr)

    e = jnp.where(sim >= thr, jnp.exp(sim), 0.0)
    z = jnp.sum(e, axis=0, keepdims=True)
    e_ref[...] = e / z


def _matmul_body(v_ref, e_ref, o_ref, acc_ref):
    k = pl.program_id(2)

    @pl.when(k == 0)
    def _():
        acc_ref[...] = jnp.zeros_like(acc_ref)

    acc_ref[...] += jnp.dot(v_ref[...], e_ref[...],
                            preferred_element_type=jnp.float32)

    @pl.when(k == pl.num_programs(2) - 1)
    def _():
        o_ref[...] = acc_ref[...]


@jax.jit
def kernel(q_key, q_selection, mem_key, mem_shrinkage, mem_value):
    qk = q_key.reshape(CK, HW)
    qs = q_selection.reshape(CK, HW)
    mkT = jnp.swapaxes(mem_key.reshape(CK, T), 0, 1)
    mk2T = mkT * mkT
    shr = mem_shrinkage.reshape(T, 1)
    v = mem_value.reshape(2 * CV, T)

    e = pl.pallas_call(
        _affinity_body,
        grid=(HW // QB,),
        in_specs=[
            pl.BlockSpec((CK, QB), lambda i: (0, i)),
            pl.BlockSpec((CK, QB), lambda i: (0, i)),
            pl.BlockSpec((T, CK), lambda i: (0, 0)),
            pl.BlockSpec((T, CK), lambda i: (0, 0)),
            pl.BlockSpec((T, 1), lambda i: (0, 0)),
        ],
        out_specs=pl.BlockSpec((T, QB), lambda i: (0, i)),
        out_shape=jax.ShapeDtypeStruct((T, HW), jnp.float32),
    )(qk, qs, mkT, mk2T, shr)

    CB, QB2, TK = 256, 256, 2048
    out = pl.pallas_call(
        _matmul_body,
        grid=(2 * CV // CB, HW // QB2, T // TK),
        in_specs=[
            pl.BlockSpec((CB, TK), lambda i, j, k: (i, k)),
            pl.BlockSpec((TK, QB2), lambda i, j, k: (k, j)),
        ],
        out_specs=pl.BlockSpec((CB, QB2), lambda i, j, k: (i, j)),
        out_shape=jax.ShapeDtypeStruct((2 * CV, HW), jnp.float32),
        scratch_shapes=[pltpu.VMEM((CB, QB2), jnp.float32)],
    )(v, e)

    return out.reshape(2, CV, H, W)


# speedup vs baseline: 9.0272x; 1.1837x over previous
import functools
import math

import jax
import jax.numpy as jnp
from jax.experimental import pallas as pl
from jax.experimental.pallas import tpu as pltpu

H = 32
W = 32
HW = H * W
T = 16384
CK = 64
CV = 512
TOP_K = 30

QB = 128
NEG = -3.0e38


def _affinity_body(qk_ref, qs_ref, mkT_ref, mk2T_ref, shr_ref, e_ref):
    qk = qk_ref[...]
    qs = qs_ref[...]
    mkT = mkT_ref[...]
    mk2T = mk2T_ref[...]
    shr = shr_ref[...]

    a_sq = jnp.dot(mk2T, qs, preferred_element_type=jnp.float32)
    two_ab = 2.0 * jnp.dot(mkT, qk * qs, preferred_element_type=jnp.float32)
    b_sq = jnp.sum(qs * (qk * qk), axis=0, keepdims=True)
    sim = (-a_sq + two_ab - b_sq) * shr * (1.0 / math.sqrt(CK))

    thr = jnp.max(sim, axis=0, keepdims=True)

    def step(_, th):
        return jnp.max(jnp.where(sim < th, sim, NEG), axis=0, keepdims=True)

    thr = jax.lax.fori_loop(0, TOP_K - 1, step, thr)

    e = jnp.where(sim >= thr, jnp.exp(sim), 0.0)
    z = jnp.sum(e, axis=0, keepdims=True)
    e_ref[...] = e / z


def _matmul_body(v_ref, e_ref, o_ref, acc_ref):
    k = pl.program_id(2)

    @pl.when(k == 0)
    def _():
        acc_ref[...] = jnp.zeros_like(acc_ref)

    acc_ref[...] += jnp.dot(v_ref[...], e_ref[...],
                            preferred_element_type=jnp.float32)

    @pl.when(k == pl.num_programs(2) - 1)
    def _():
        o_ref[...] = acc_ref[...]


@jax.jit
def kernel(q_key, q_selection, mem_key, mem_shrinkage, mem_value):
    qk = q_key.reshape(CK, HW)
    qs = q_selection.reshape(CK, HW)
    mkT = jnp.swapaxes(mem_key.reshape(CK, T), 0, 1)
    mk2T = mkT * mkT
    shr = mem_shrinkage.reshape(T, 1)
    v = mem_value.reshape(2 * CV, T)

    e = pl.pallas_call(
        _affinity_body,
        grid=(HW // QB,),
        in_specs=[
            pl.BlockSpec((CK, QB), lambda i: (0, i)),
            pl.BlockSpec((CK, QB), lambda i: (0, i)),
            pl.BlockSpec((T, CK), lambda i: (0, 0)),
            pl.BlockSpec((T, CK), lambda i: (0, 0)),
            pl.BlockSpec((T, 1), lambda i: (0, 0)),
        ],
        out_specs=pl.BlockSpec((T, QB), lambda i: (0, i)),
        out_shape=jax.ShapeDtypeStruct((T, HW), jnp.float32),
    )(qk, qs, mkT, mk2T, shr)

    CB, QB2, TK = 512, 1024, 2048
    out = pl.pallas_call(
        _matmul_body,
        grid=(2 * CV // CB, HW // QB2, T // TK),
        in_specs=[
            pl.BlockSpec((CB, TK), lambda i, j, k: (i, k)),
            pl.BlockSpec((TK, QB2), lambda i, j, k: (k, j)),
        ],
        out_specs=pl.BlockSpec((CB, QB2), lambda i, j, k: (i, j)),
        out_shape=jax.ShapeDtypeStruct((2 * CV, HW), jnp.float32),
        scratch_shapes=[pltpu.VMEM((CB, QB2), jnp.float32)],
    )(v, e)

    return out.reshape(2, CV, H, W)
